# Initial kernel scaffold; baseline (speedup 1.0000x reference)
import jax
import jax.numpy as jnp
from jax.experimental import pallas as pl


def kernel(image_latent, base_watermark, input_conv__w0, input_conv__b0, down1__w0, down1__w1, down1__w2, down1__b0, down1__b1, down1__b2, down2__w0, down2__w1, down2__w2, down2__b0, down2__b1, down2__b2, down3__w0, down3__w1, down3__w2, down3__b0, down3__b1, down3__b2, bottleneck__w0, bottleneck__w1, bottleneck__w2, bottleneck__b0, bottleneck__b1, bottleneck__b2, up3__w0, up3__b0, att3__w0, att3__w1, att3__b0, att3__b1, att3__ws, att3__bs, up2__w0, up2__b0, att2__w0, att2__w1, att2__b0, att2__b1, att2__ws, att2__bs, up1__w0, up1__b0, att1__w0, att1__w1, att1__b0, att1__b1, att1__ws, att1__bs, out__w, out__b):
    raise NotImplementedError("write your pallas kernel here")


# speedup vs baseline: 1.1977x; 1.1977x over previous
import functools

import jax
import jax.numpy as jnp
from jax import lax
from jax.experimental import pallas as pl
from jax.experimental.pallas import tpu as pltpu

_EPS = 1e-5


def _inorm_relu(y):
    m = jnp.mean(y, axis=(1, 2), keepdims=True)
    v = jnp.mean(y * y, axis=(1, 2), keepdims=True) - m * m
    return jnp.maximum((y - m) * lax.rsqrt(v + _EPS), 0.0)


def _pad1(y):
    return jnp.pad(y, ((0, 0), (1, 1), (1, 1), (0, 0)))


def _conv9(xp, w9):
    b, hp, wp, c = xp.shape
    h, w = hp - 2, wp - 2
    taps = [xp[:, di:di + h, dj:dj + w, :]
            for di in range(3) for dj in range(3)]
    lhs = jnp.concatenate(taps, axis=-1).reshape(b * h * w, 9 * c)
    y = jnp.dot(lhs, w9, preferred_element_type=jnp.float32)
    return y.reshape(b, h, w, w9.shape[1])


def _conv9_s2(xref, w9):
    b, hp, wp, c = xref.shape
    ho, wo = (hp - 2) // 2, (wp - 2) // 2
    taps = [xref[:, pl.ds(di, ho, 2), pl.ds(dj, wo, 2), :]
            for di in range(3) for dj in range(3)]
    lhs = jnp.concatenate(taps, axis=-1).reshape(b * ho * wo, 9 * c)
    y = jnp.dot(lhs, w9, preferred_element_type=jnp.float32)
    return y.reshape(b, ho, wo, w9.shape[1])


def _down_body(x_ref, w_ic, wd10, wd11, wd12, wd20, wd21, wd22,
               wd30, wd31, wd32, wb0, wb1, wb2,
               x0_ref, d1_ref, d2_ref, bn_ref):
    xp = x_ref[...]
    x0_ref[...] = _pad1(_inorm_relu(_conv9(xp, w_ic[...])))
    y = _inorm_relu(_conv9_s2(x0_ref, wd10[...]))
    y = _inorm_relu(_conv9(_pad1(y), wd11[...]))
    d1_ref[...] = _pad1(_inorm_relu(_conv9(_pad1(y), wd12[...])))
    y = _inorm_relu(_conv9_s2(d1_ref, wd20[...]))
    y = _inorm_relu(_conv9(_pad1(y), wd21[...]))
    d2_ref[...] = _pad1(_inorm_relu(_conv9(_pad1(y), wd22[...])))
    y = _inorm_relu(_conv9_s2(d2_ref, wd30[...]))
    y = _inorm_relu(_conv9(_pad1(y), wd31[...]))
    d3 = _inorm_relu(_conv9(_pad1(y), wd32[...]))
    y = _inorm_relu(_conv9(_pad1(d3), wb0[...]))
    y = _inorm_relu(_conv9(_pad1(y), wb1[...]))
    y = _inorm_relu(_conv9(_pad1(y), wb2[...]))
    bn_ref[...] = y + d3


def _att_body(u_ref, s_ref, wu, wa0, wa1, ws, bs, o_ref):
    y = _inorm_relu(_conv9(u_ref[...], wu[...]))
    z = jnp.concatenate([s_ref[...], _pad1(y)], axis=-1)
    b, hp, wp, cz = z.shape
    h1 = _inorm_relu(_conv9(z, wa0[...]))
    h2 = _inorm_relu(_conv9(_pad1(h1), wa1[...]))
    zv = z[:, 1:hp - 1, 1:wp - 1, :].reshape(b * (hp - 2) * (wp - 2), cz)
    res = jnp.dot(zv, ws[...], preferred_element_type=jnp.float32) + bs[...]
    o_ref[...] = h2 + res.reshape(h2.shape)


def _final_body(u_ref, s_ref, lat_ref, wu, wa0, wa1, ws, bs, wo, bo, o_ref):
    y = _inorm_relu(_conv9(u_ref[...], wu[...]))
    z = jnp.concatenate([s_ref[...], _pad1(y)], axis=-1)
    b, hp, wp, cz = z.shape
    h, w = hp - 2, wp - 2
    h1 = _inorm_relu(_conv9(z, wa0[...]))
    h2 = _inorm_relu(_conv9(_pad1(h1), wa1[...]))
    zv = z[:, 1:hp - 1, 1:wp - 1, :].reshape(b * h * w, cz)
    res = jnp.dot(zv, ws[...], preferred_element_type=jnp.float32) + bs[...]
    u1 = h2.reshape(b * h * w, h2.shape[-1]) + res
    out = jnp.dot(u1, wo[...], preferred_element_type=jnp.float32) + bo[...]
    o_ref[...] = out.reshape(b, h, w, wo.shape[1]) + lat_ref[...]


def _spec_batch(shape, b):
    nd = len(shape)
    return pl.BlockSpec((b,) + tuple(shape[1:]),
                        lambda i, _n=nd: (i,) + (0,) * (_n - 1))


def _spec_full(shape):
    nd = len(shape)
    return pl.BlockSpec(tuple(shape), lambda i, _n=nd: (0,) * _n)


def _w9(w):
    return jnp.transpose(w, (2, 3, 1, 0)).reshape(9 * w.shape[1], w.shape[0])


def _up2(x):
    for ax in (1, 2):
        n = x.shape[ax]
        lo = jnp.concatenate([lax.slice_in_dim(x, 0, 1, axis=ax),
                              lax.slice_in_dim(x, 0, n - 1, axis=ax)], axis=ax)
        hi = jnp.concatenate([lax.slice_in_dim(x, 1, n, axis=ax),
                              lax.slice_in_dim(x, n - 1, n, axis=ax)], axis=ax)
        ev = 0.75 * x + 0.25 * lo
        od = 0.75 * x + 0.25 * hi
        x = jnp.stack([ev, od], axis=ax + 1).reshape(
            x.shape[:ax] + (2 * n,) + x.shape[ax + 1:])
    return x


def _pick_b(n, pref):
    for b in range(min(pref, n), 0, -1):
        if n % b == 0:
            return b
    return 1


def _run(body, ins, batched, b, out_shapes):
    n = ins[0].shape[0]
    specs = [_spec_batch(a.shape, b) if k else _spec_full(a.shape)
             for a, k in zip(ins, batched)]
    out_specs = [_spec_batch(s.shape, b) for s in out_shapes]
    outs = pl.pallas_call(
        body,
        grid=(n // b,),
        in_specs=specs,
        out_shape=out_shapes,
        out_specs=out_specs,
        compiler_params=pltpu.CompilerParams(
            dimension_semantics=("parallel",)),
    )(*ins)
    return outs


def kernel(image_latent, base_watermark, input_conv__w0, input_conv__b0, down1__w0, down1__w1, down1__w2, down1__b0, down1__b1, down1__b2, down2__w0, down2__w1, down2__w2, down2__b0, down2__b1, down2__b2, down3__w0, down3__w1, down3__w2, down3__b0, down3__b1, down3__b2, bottleneck__w0, bottleneck__w1, bottleneck__w2, bottleneck__b0, bottleneck__b1, bottleneck__b2, up3__w0, up3__b0, att3__w0, att3__w1, att3__b0, att3__b1, att3__ws, att3__bs, up2__w0, up2__b0, att2__w0, att2__w1, att2__b0, att2__b1, att2__ws, att2__bs, up1__w0, up1__b0, att1__w0, att1__w1, att1__b0, att1__b1, att1__ws, att1__bs, out__w, out__b):
    n, _, h, w = image_latent.shape
    lat = jnp.transpose(image_latent, (0, 2, 3, 1))
    wmk = jnp.transpose(base_watermark, (0, 2, 3, 1))
    x_in = _pad1(jnp.concatenate([lat, wmk], axis=-1))

    c8 = x_in.shape[-1]
    hd4 = input_conv__w0.shape[0]
    hd2 = down1__w1.shape[0]
    hd = down2__w1.shape[0]
    hd2x = down3__w1.shape[0]

    dws = [_w9(v) for v in
           (input_conv__w0, down1__w0, down1__w1, down1__w2,
            down2__w0, down2__w1, down2__w2,
            down3__w0, down3__w1, down3__w2,
            bottleneck__w0, bottleneck__w1, bottleneck__w2)]
    f32 = jnp.float32
    b1 = _pick_b(n, 1)
    x0p, d1p, d2p, bn = _run(
        _down_body, [x_in] + dws, [True] + [False] * 13, b1,
        [jax.ShapeDtypeStruct((n, h + 2, w + 2, hd4), f32),
         jax.ShapeDtypeStruct((n, h // 2 + 2, w // 2 + 2, hd2), f32),
         jax.ShapeDtypeStruct((n, h // 4 + 2, w // 4 + 2, hd), f32),
         jax.ShapeDtypeStruct((n, h // 8, w // 8, hd2x), f32)])

    def att_stage(up_in, skip, wu, wa0, wa1, ws, bs, cout, b):
        hh = up_in.shape[1] - 2
        ins = [up_in, skip, _w9(wu), _w9(wa0), _w9(wa1),
               ws[:, :, 0, 0].T, bs.reshape(1, -1)]
        return _run(_att_body, ins, [True, True] + [False] * 5, b,
                    [jax.ShapeDtypeStruct((n, hh, hh, cout), f32)])[0]

    u3 = att_stage(_pad1(_up2(bn)), d2p, up3__w0, att3__w0, att3__w1,
                   att3__ws, att3__bs, hd, _pick_b(n, 8))
    u2 = att_stage(_pad1(_up2(u3)), d1p, up2__w0, att2__w0, att2__w1,
                   att2__ws, att2__bs, hd2, _pick_b(n, 4))

    fins = [_pad1(_up2(u2)), x0p, lat, _w9(up1__w0), _w9(att1__w0),
            _w9(att1__w1), att1__ws[:, :, 0, 0].T, att1__bs.reshape(1, -1),
            out__w[:, :, 0, 0].T, out__b.reshape(1, -1)]
    out = _run(_final_body, fins, [True, True, True] + [False] * 7,
               _pick_b(n, 1),
               [jax.ShapeDtypeStruct((n, h, w, out__w.shape[0]), f32)])[0]
    return jnp.transpose(out, (0, 3, 1, 2))


# speedup vs baseline: 1.7843x; 1.4898x over previous
import functools

import jax
import jax.numpy as jnp
from jax import lax
from jax.experimental import pallas as pl
from jax.experimental.pallas import tpu as pltpu

_EPS = 1e-5


def _inorm_relu(y):
    m = jnp.mean(y, axis=(1, 2), keepdims=True)
    v = jnp.mean(y * y, axis=(1, 2), keepdims=True) - m * m
    return jnp.maximum((y - m) * lax.rsqrt(v + _EPS), 0.0)


def _pad1(y):
    return jnp.pad(y, ((0, 0), (1, 1), (1, 1), (0, 0)))


def _conv9(xp, w9):
    b, hp, wp, c = xp.shape
    h, w = hp - 2, wp - 2
    taps = [xp[:, di:di + h, dj:dj + w, :]
            for di in range(3) for dj in range(3)]
    lhs = jnp.concatenate(taps, axis=-1).reshape(b * h * w, 9 * c)
    y = jnp.dot(lhs, w9, preferred_element_type=jnp.float32)
    return y.reshape(b, h, w, w9.shape[1])


def _conv9_s2(xref, w9):
    b, hp, wp, c = xref.shape
    ho, wo = (hp - 2) // 2, (wp - 2) // 2
    taps = [xref[:, pl.ds(di, ho, 2), pl.ds(dj, wo, 2), :]
            for di in range(3) for dj in range(3)]
    lhs = jnp.concatenate(taps, axis=-1).reshape(b * ho * wo, 9 * c)
    y = jnp.dot(lhs, w9, preferred_element_type=jnp.float32)
    return y.reshape(b, ho, wo, w9.shape[1])


def _to_rows(x_cl, h, w):
    xt = x_cl.T.reshape(1, h, w, x_cl.shape[0])
    return _pad1(xt)


def _down_body(lat_ref, wm_ref, w_ic, wd10, wd11, wd12, wd20, wd21, wd22,
               wd30, wd31, wd32, wb0, wb1, wb2,
               x0_ref, d1_ref, d2_ref, bn_ref):
    h, w = x0_ref.shape[1] - 2, x0_ref.shape[2] - 2
    xc = jnp.concatenate([lat_ref[0], wm_ref[0]], axis=0)
    xp = _to_rows(xc, h, w)
    x0_ref[...] = _pad1(_inorm_relu(_conv9(xp, w_ic[...])))
    y = _inorm_relu(_conv9_s2(x0_ref, wd10[...]))
    y = _inorm_relu(_conv9(_pad1(y), wd11[...]))
    d1_ref[...] = _pad1(_inorm_relu(_conv9(_pad1(y), wd12[...])))
    y = _inorm_relu(_conv9_s2(d1_ref, wd20[...]))
    y = _inorm_relu(_conv9(_pad1(y), wd21[...]))
    d2_ref[...] = _pad1(_inorm_relu(_conv9(_pad1(y), wd22[...])))
    y = _inorm_relu(_conv9_s2(d2_ref, wd30[...]))
    y = _inorm_relu(_conv9(_pad1(y), wd31[...]))
    d3 = _inorm_relu(_conv9(_pad1(y), wd32[...]))
    y = _inorm_relu(_conv9(_pad1(d3), wb0[...]))
    y = _inorm_relu(_conv9(_pad1(y), wb1[...]))
    y = _inorm_relu(_conv9(_pad1(y), wb2[...]))
    bn_ref[...] = y + d3


def _att_body(u_ref, s_ref, wu, wa0, wa1, ws, bs, o_ref):
    y = _inorm_relu(_conv9(u_ref[...], wu[...]))
    z = jnp.concatenate([s_ref[...], _pad1(y)], axis=-1)
    b, hp, wp, cz = z.shape
    h1 = _inorm_relu(_conv9(z, wa0[...]))
    h2 = _inorm_relu(_conv9(_pad1(h1), wa1[...]))
    zv = z[:, 1:hp - 1, 1:wp - 1, :].reshape(b * (hp - 2) * (wp - 2), cz)
    res = jnp.dot(zv, ws[...], preferred_element_type=jnp.float32) + bs[...]
    o_ref[...] = h2 + res.reshape(h2.shape)


def _final_body(u_ref, s_ref, lat_ref, wu, wa0, wa1, ws, bs, wo, bo, o_ref):
    y = _inorm_relu(_conv9(u_ref[...], wu[...]))
    z = jnp.concatenate([s_ref[...], _pad1(y)], axis=-1)
    b, hp, wp, cz = z.shape
    h, w = hp - 2, wp - 2
    h1 = _inorm_relu(_conv9(z, wa0[...]))
    h2 = _inorm_relu(_conv9(_pad1(h1), wa1[...]))
    zv = z[:, 1:hp - 1, 1:wp - 1, :].reshape(b * h * w, cz)
    res = jnp.dot(zv, ws[...], preferred_element_type=jnp.float32) + bs[...]
    u1 = h2.reshape(b * h * w, h2.shape[-1]) + res
    out = jnp.dot(u1, wo[...], preferred_element_type=jnp.float32) + bo[...]
    o_ref[...] = (out.T + lat_ref[0])[None]


def _spec_batch(shape, b):
    nd = len(shape)
    return pl.BlockSpec((b,) + tuple(shape[1:]),
                        lambda i, _n=nd: (i,) + (0,) * (_n - 1))


def _spec_full(shape):
    nd = len(shape)
    return pl.BlockSpec(tuple(shape), lambda i, _n=nd: (0,) * _n)


def _w9(w):
    return jnp.transpose(w, (2, 3, 1, 0)).reshape(9 * w.shape[1], w.shape[0])


def _up2(x):
    for ax in (1, 2):
        n = x.shape[ax]
        lo = jnp.concatenate([lax.slice_in_dim(x, 0, 1, axis=ax),
                              lax.slice_in_dim(x, 0, n - 1, axis=ax)], axis=ax)
        hi = jnp.concatenate([lax.slice_in_dim(x, 1, n, axis=ax),
                              lax.slice_in_dim(x, n - 1, n, axis=ax)], axis=ax)
        ev = 0.75 * x + 0.25 * lo
        od = 0.75 * x + 0.25 * hi
        x = jnp.stack([ev, od], axis=ax + 1).reshape(
            x.shape[:ax] + (2 * n,) + x.shape[ax + 1:])
    return x


def _pick_b(n, pref):
    for b in range(min(pref, n), 0, -1):
        if n % b == 0:
            return b
    return 1


def _run(body, ins, batched, b, out_shapes):
    n = ins[0].shape[0]
    specs = [_spec_batch(a.shape, b) if k else _spec_full(a.shape)
             for a, k in zip(ins, batched)]
    out_specs = [_spec_batch(s.shape, b) for s in out_shapes]
    outs = pl.pallas_call(
        body,
        grid=(n // b,),
        in_specs=specs,
        out_shape=out_shapes,
        out_specs=out_specs,
        compiler_params=pltpu.CompilerParams(
            dimension_semantics=("parallel",)),
    )(*ins)
    return outs


def kernel(image_latent, base_watermark, input_conv__w0, input_conv__b0, down1__w0, down1__w1, down1__w2, down1__b0, down1__b1, down1__b2, down2__w0, down2__w1, down2__w2, down2__b0, down2__b1, down2__b2, down3__w0, down3__w1, down3__w2, down3__b0, down3__b1, down3__b2, bottleneck__w0, bottleneck__w1, bottleneck__w2, bottleneck__b0, bottleneck__b1, bottleneck__b2, up3__w0, up3__b0, att3__w0, att3__w1, att3__b0, att3__b1, att3__ws, att3__bs, up2__w0, up2__b0, att2__w0, att2__w1, att2__b0, att2__b1, att2__ws, att2__bs, up1__w0, up1__b0, att1__w0, att1__w1, att1__b0, att1__b1, att1__ws, att1__bs, out__w, out__b):
    n, cl, h, w = image_latent.shape
    lat = image_latent.reshape(n, cl, h * w)
    wmk = base_watermark.reshape(n, cl, h * w)

    hd4 = input_conv__w0.shape[0]
    hd2 = down1__w1.shape[0]
    hd = down2__w1.shape[0]
    hd2x = down3__w1.shape[0]

    dws = [_w9(v) for v in
           (input_conv__w0, down1__w0, down1__w1, down1__w2,
            down2__w0, down2__w1, down2__w2,
            down3__w0, down3__w1, down3__w2,
            bottleneck__w0, bottleneck__w1, bottleneck__w2)]
    f32 = jnp.float32
    b1 = _pick_b(n, 1)
    x0p, d1p, d2p, bn = _run(
        _down_body, [lat, wmk] + dws, [True, True] + [False] * 13, b1,
        [jax.ShapeDtypeStruct((n, h + 2, w + 2, hd4), f32),
         jax.ShapeDtypeStruct((n, h // 2 + 2, w // 2 + 2, hd2), f32),
         jax.ShapeDtypeStruct((n, h // 4 + 2, w // 4 + 2, hd), f32),
         jax.ShapeDtypeStruct((n, h // 8, w // 8, hd2x), f32)])

    def att_stage(up_in, skip, wu, wa0, wa1, ws, bs, cout, b):
        hh = up_in.shape[1] - 2
        ins = [up_in, skip, _w9(wu), _w9(wa0), _w9(wa1),
               ws[:, :, 0, 0].T, bs.reshape(1, -1)]
        return _run(_att_body, ins, [True, True] + [False] * 5, b,
                    [jax.ShapeDtypeStruct((n, hh, hh, cout), f32)])[0]

    u3 = att_stage(_pad1(_up2(bn)), d2p, up3__w0, att3__w0, att3__w1,
                   att3__ws, att3__bs, hd, _pick_b(n, 8))
    u2 = att_stage(_pad1(_up2(u3)), d1p, up2__w0, att2__w0, att2__w1,
                   att2__ws, att2__bs, hd2, _pick_b(n, 4))

    fins = [_pad1(_up2(u2)), x0p, lat, _w9(up1__w0), _w9(att1__w0),
            _w9(att1__w1), att1__ws[:, :, 0, 0].T, att1__bs.reshape(1, -1),
            out__w[:, :, 0, 0].T, out__b.reshape(1, -1)]
    out = _run(_final_body, fins, [True, True, True] + [False] * 7,
               _pick_b(n, 1),
               [jax.ShapeDtypeStruct((n, out__w.shape[0], h * w), f32)])[0]
    return out.reshape(n, out__w.shape[0], h, w)


# speedup vs baseline: 1.8560x; 1.0402x over previous
import functools

import jax
import jax.numpy as jnp
from jax import lax
from jax.experimental import pallas as pl
from jax.experimental.pallas import tpu as pltpu

_EPS = 1e-5


def _inorm_relu(y):
    m = jnp.mean(y, axis=(1, 2), keepdims=True)
    v = jnp.mean(y * y, axis=(1, 2), keepdims=True) - m * m
    return jnp.maximum((y - m) * lax.rsqrt(v + _EPS), 0.0)


def _pad1(y):
    return jnp.pad(y, ((0, 0), (1, 1), (1, 1), (0, 0)))


def _conv9(xp, w9):
    b, hp, wp, c = xp.shape
    h, w = hp - 2, wp - 2
    taps = [xp[:, di:di + h, dj:dj + w, :]
            for di in range(3) for dj in range(3)]
    lhs = jnp.concatenate(taps, axis=-1).reshape(b * h * w, 9 * c)
    y = jnp.dot(lhs, w9, preferred_element_type=jnp.float32)
    return y.reshape(b, h, w, w9.shape[1])


def _conv9_s2(xref, w9):
    b, hp, wp, c = xref.shape
    ho, wo = (hp - 2) // 2, (wp - 2) // 2
    taps = [xref[:, pl.ds(di, ho, 2), pl.ds(dj, wo, 2), :]
            for di in range(3) for dj in range(3)]
    lhs = jnp.concatenate(taps, axis=-1).reshape(b * ho * wo, 9 * c)
    y = jnp.dot(lhs, w9, preferred_element_type=jnp.float32)
    return y.reshape(b, ho, wo, w9.shape[1])


def _to_rows(x_cl, h, w):
    b, c, _ = x_cl.shape
    xt = jnp.swapaxes(x_cl, 1, 2).reshape(b, h, w, c)
    return _pad1(xt)


def _down_body(lat_ref, wm_ref, w_ic, wd10, wd11, wd12, wd20, wd21, wd22,
               wd30, wd31, wd32, wb0, wb1, wb2,
               x0_ref, d1_ref, d2_ref, bn_ref):
    h, w = x0_ref.shape[1] - 2, x0_ref.shape[2] - 2
    xc = jnp.concatenate([lat_ref[...], wm_ref[...]], axis=1)
    xp = _to_rows(xc, h, w)
    x0_ref[...] = _pad1(_inorm_relu(_conv9(xp, w_ic[...])))
    y = _inorm_relu(_conv9_s2(x0_ref, wd10[...]))
    y = _inorm_relu(_conv9(_pad1(y), wd11[...]))
    d1_ref[...] = _pad1(_inorm_relu(_conv9(_pad1(y), wd12[...])))
    y = _inorm_relu(_conv9_s2(d1_ref, wd20[...]))
    y = _inorm_relu(_conv9(_pad1(y), wd21[...]))
    d2_ref[...] = _pad1(_inorm_relu(_conv9(_pad1(y), wd22[...])))
    y = _inorm_relu(_conv9_s2(d2_ref, wd30[...]))
    y = _inorm_relu(_conv9(_pad1(y), wd31[...]))
    d3 = _inorm_relu(_conv9(_pad1(y), wd32[...]))
    y = _inorm_relu(_conv9(_pad1(d3), wb0[...]))
    y = _inorm_relu(_conv9(_pad1(y), wb1[...]))
    y = _inorm_relu(_conv9(_pad1(y), wb2[...]))
    bn_ref[...] = y + d3


def _att_body(u_ref, s_ref, wu, wa0, wa1, ws, bs, o_ref):
    y = _inorm_relu(_conv9(u_ref[...], wu[...]))
    z = jnp.concatenate([s_ref[...], _pad1(y)], axis=-1)
    b, hp, wp, cz = z.shape
    h1 = _inorm_relu(_conv9(z, wa0[...]))
    h2 = _inorm_relu(_conv9(_pad1(h1), wa1[...]))
    zv = z[:, 1:hp - 1, 1:wp - 1, :].reshape(b * (hp - 2) * (wp - 2), cz)
    res = jnp.dot(zv, ws[...], preferred_element_type=jnp.float32) + bs[...]
    o_ref[...] = h2 + res.reshape(h2.shape)


def _final_body(u_ref, s_ref, lat_ref, wu, wa0, wa1, ws, bs, wo, bo, o_ref):
    y = _inorm_relu(_conv9(u_ref[...], wu[...]))
    z = jnp.concatenate([s_ref[...], _pad1(y)], axis=-1)
    b, hp, wp, cz = z.shape
    h, w = hp - 2, wp - 2
    h1 = _inorm_relu(_conv9(z, wa0[...]))
    h2 = _inorm_relu(_conv9(_pad1(h1), wa1[...]))
    zv = z[:, 1:hp - 1, 1:wp - 1, :].reshape(b * h * w, cz)
    res = jnp.dot(zv, ws[...], preferred_element_type=jnp.float32) + bs[...]
    u1 = h2.reshape(b * h * w, h2.shape[-1]) + res
    out = jnp.dot(u1, wo[...], preferred_element_type=jnp.float32) + bo[...]
    out = jnp.swapaxes(out.reshape(b, h * w, wo.shape[1]), 1, 2)
    o_ref[...] = out + lat_ref[...]


def _spec_batch(shape, b):
    nd = len(shape)
    return pl.BlockSpec((b,) + tuple(shape[1:]),
                        lambda i, _n=nd: (i,) + (0,) * (_n - 1))


def _spec_full(shape):
    nd = len(shape)
    return pl.BlockSpec(tuple(shape), lambda i, _n=nd: (0,) * _n)


def _w9(w):
    return jnp.transpose(w, (2, 3, 1, 0)).reshape(9 * w.shape[1], w.shape[0])


def _up2(x):
    for ax in (1, 2):
        n = x.shape[ax]
        lo = jnp.concatenate([lax.slice_in_dim(x, 0, 1, axis=ax),
                              lax.slice_in_dim(x, 0, n - 1, axis=ax)], axis=ax)
        hi = jnp.concatenate([lax.slice_in_dim(x, 1, n, axis=ax),
                              lax.slice_in_dim(x, n - 1, n, axis=ax)], axis=ax)
        ev = 0.75 * x + 0.25 * lo
        od = 0.75 * x + 0.25 * hi
        x = jnp.stack([ev, od], axis=ax + 1).reshape(
            x.shape[:ax] + (2 * n,) + x.shape[ax + 1:])
    return x


def _pick_b(n, pref):
    for b in range(min(pref, n), 0, -1):
        if n % b == 0:
            return b
    return 1


def _run(body, ins, batched, b, out_shapes):
    n = ins[0].shape[0]
    specs = [_spec_batch(a.shape, b) if k else _spec_full(a.shape)
             for a, k in zip(ins, batched)]
    out_specs = [_spec_batch(s.shape, b) for s in out_shapes]
    outs = pl.pallas_call(
        body,
        grid=(n // b,),
        in_specs=specs,
        out_shape=out_shapes,
        out_specs=out_specs,
        compiler_params=pltpu.CompilerParams(
            dimension_semantics=("parallel",)),
    )(*ins)
    return outs


def kernel(image_latent, base_watermark, input_conv__w0, input_conv__b0, down1__w0, down1__w1, down1__w2, down1__b0, down1__b1, down1__b2, down2__w0, down2__w1, down2__w2, down2__b0, down2__b1, down2__b2, down3__w0, down3__w1, down3__w2, down3__b0, down3__b1, down3__b2, bottleneck__w0, bottleneck__w1, bottleneck__w2, bottleneck__b0, bottleneck__b1, bottleneck__b2, up3__w0, up3__b0, att3__w0, att3__w1, att3__b0, att3__b1, att3__ws, att3__bs, up2__w0, up2__b0, att2__w0, att2__w1, att2__b0, att2__b1, att2__ws, att2__bs, up1__w0, up1__b0, att1__w0, att1__w1, att1__b0, att1__b1, att1__ws, att1__bs, out__w, out__b):
    n, cl, h, w = image_latent.shape
    lat = image_latent.reshape(n, cl, h * w)
    wmk = base_watermark.reshape(n, cl, h * w)

    hd4 = input_conv__w0.shape[0]
    hd2 = down1__w1.shape[0]
    hd = down2__w1.shape[0]
    hd2x = down3__w1.shape[0]

    dws = [_w9(v) for v in
           (input_conv__w0, down1__w0, down1__w1, down1__w2,
            down2__w0, down2__w1, down2__w2,
            down3__w0, down3__w1, down3__w2,
            bottleneck__w0, bottleneck__w1, bottleneck__w2)]
    f32 = jnp.float32
    b1 = _pick_b(n, 2)
    x0p, d1p, d2p, bn = _run(
        _down_body, [lat, wmk] + dws, [True, True] + [False] * 13, b1,
        [jax.ShapeDtypeStruct((n, h + 2, w + 2, hd4), f32),
         jax.ShapeDtypeStruct((n, h // 2 + 2, w // 2 + 2, hd2), f32),
         jax.ShapeDtypeStruct((n, h // 4 + 2, w // 4 + 2, hd), f32),
         jax.ShapeDtypeStruct((n, h // 8, w // 8, hd2x), f32)])

    def att_stage(up_in, skip, wu, wa0, wa1, ws, bs, cout, b):
        hh = up_in.shape[1] - 2
        ins = [up_in, skip, _w9(wu), _w9(wa0), _w9(wa1),
               ws[:, :, 0, 0].T, bs.reshape(1, -1)]
        return _run(_att_body, ins, [True, True] + [False] * 5, b,
                    [jax.ShapeDtypeStruct((n, hh, hh, cout), f32)])[0]

    u3 = att_stage(_pad1(_up2(bn)), d2p, up3__w0, att3__w0, att3__w1,
                   att3__ws, att3__bs, hd, _pick_b(n, 8))
    u2 = att_stage(_pad1(_up2(u3)), d1p, up2__w0, att2__w0, att2__w1,
                   att2__ws, att2__bs, hd2, _pick_b(n, 4))

    fins = [_pad1(_up2(u2)), x0p, lat, _w9(up1__w0), _w9(att1__w0),
            _w9(att1__w1), att1__ws[:, :, 0, 0].T, att1__bs.reshape(1, -1),
            out__w[:, :, 0, 0].T, out__b.reshape(1, -1)]
    out = _run(_final_body, fins, [True, True, True] + [False] * 7,
               _pick_b(n, 2),
               [jax.ShapeDtypeStruct((n, out__w.shape[0], h * w), f32)])[0]
    return out.reshape(n, out__w.shape[0], h, w)


# speedup vs baseline: 3.7032x; 1.9953x over previous
import functools

import jax
import jax.numpy as jnp
from jax import lax
from jax.experimental import pallas as pl
from jax.experimental.pallas import tpu as pltpu

_EPS = 1e-5


def _inorm_relu(y):
    m = jnp.mean(y, axis=(1, 2), keepdims=True)
    v = jnp.mean(y * y, axis=(1, 2), keepdims=True) - m * m
    return jnp.maximum((y - m) * lax.rsqrt(v + _EPS), 0.0)


def _pad1(y):
    return jnp.pad(y, ((0, 0), (1, 1), (1, 1), (0, 0)))


def _conv9(xp, w9):
    b, hp, wp, c = xp.shape
    h, w = hp - 2, wp - 2
    taps = [xp[:, di:di + h, dj:dj + w, :]
            for di in range(3) for dj in range(3)]
    lhs = jnp.concatenate(taps, axis=-1).reshape(b * h * w, 9 * c)
    y = jnp.dot(lhs, w9, preferred_element_type=jnp.float32)
    return y.reshape(b, h, w, w9.shape[1])


def _conv9_s2(xref, w9):
    b, hp, wp, c = xref.shape
    ho, wo = (hp - 2) // 2, (wp - 2) // 2
    taps = [xref[:, pl.ds(di, ho, 2), pl.ds(dj, wo, 2), :]
            for di in range(3) for dj in range(3)]
    lhs = jnp.concatenate(taps, axis=-1).reshape(b * ho * wo, 9 * c)
    y = jnp.dot(lhs, w9, preferred_element_type=jnp.float32)
    return y.reshape(b, ho, wo, w9.shape[1])


def _to_rows(x_cl, h, w):
    b, c, _ = x_cl.shape
    xt = jnp.swapaxes(x_cl, 1, 2).reshape(b, h, w, c)
    return _pad1(xt)


def _cm(y):
    b, hh, ww, c = y.shape
    return jnp.swapaxes(y.reshape(b, hh * ww, c), 1, 2)


def _inorm_relu_t(y):
    m = jnp.mean(y, axis=2, keepdims=True)
    v = jnp.mean(y * y, axis=2, keepdims=True) - m * m
    return jnp.maximum((y - m) * lax.rsqrt(v + _EPS), 0.0)


def _shift_p(x, off):
    if off == 0:
        return x
    p = x.shape[-1]
    if off > 0:
        z = jnp.zeros(x.shape[:-1] + (off,), x.dtype)
        return jnp.concatenate([x[..., off:], z], axis=-1)
    z = jnp.zeros(x.shape[:-1] + (-off,), x.dtype)
    return jnp.concatenate([z, x[..., :p + off]], axis=-1)


def _conv9_t(x, w9t, wdim):
    b, c, p = x.shape
    col = lax.broadcasted_iota(jnp.int32, (1, 1, p), 2) % wdim
    taps = []
    for di in range(3):
        for dj in range(3):
            t = _shift_p(x, (di - 1) * wdim + (dj - 1))
            if dj == 0:
                t = jnp.where(col > 0, t, 0.0)
            elif dj == 2:
                t = jnp.where(col < wdim - 1, t, 0.0)
            taps.append(t)
    lhs = jnp.concatenate(taps, axis=1)
    return jnp.stack([jnp.dot(w9t[...], lhs[bi],
                              preferred_element_type=jnp.float32)
                      for bi in range(b)], axis=0)


def _down_body(lat_ref, wm_ref, w_ic, wd10, wd11, wd12, wd20, wd21, wd22,
               wd30, wd31, wd32, wb0, wb1, wb2,
               x0c_ref, d1c_ref, d2c_ref, bn_ref,
               x0p_ref, d1p_ref, d2p_ref):
    h, w = x0p_ref.shape[1] - 2, x0p_ref.shape[2] - 2
    xc = jnp.concatenate([lat_ref[...], wm_ref[...]], axis=1)
    xp = _to_rows(xc, h, w)
    y0 = _inorm_relu(_conv9(xp, w_ic[...]))
    x0p_ref[...] = _pad1(y0)
    x0c_ref[...] = _cm(y0)
    y = _inorm_relu(_conv9_s2(x0p_ref, wd10[...]))
    y = _inorm_relu(_conv9(_pad1(y), wd11[...]))
    y = _inorm_relu(_conv9(_pad1(y), wd12[...]))
    d1p_ref[...] = _pad1(y)
    d1c_ref[...] = _cm(y)
    y = _inorm_relu(_conv9_s2(d1p_ref, wd20[...]))
    y = _inorm_relu(_conv9(_pad1(y), wd21[...]))
    y = _inorm_relu(_conv9(_pad1(y), wd22[...]))
    d2p_ref[...] = _pad1(y)
    d2c_ref[...] = _cm(y)
    y = _inorm_relu(_conv9_s2(d2p_ref, wd30[...]))
    y = _inorm_relu(_conv9(_pad1(y), wd31[...]))
    d3 = _inorm_relu(_conv9(_pad1(y), wd32[...]))
    y = _inorm_relu(_conv9(_pad1(d3), wb0[...]))
    y = _inorm_relu(_conv9(_pad1(y), wb1[...]))
    y = _inorm_relu(_conv9(_pad1(y), wb2[...]))
    bn_ref[...] = _cm(y + d3)


def _att_body_t(u_ref, s_ref, wu, wa0, wa1, ws, bs, o_ref, *, wdim):
    y = _inorm_relu_t(_conv9_t(u_ref[...], wu, wdim))
    z = jnp.concatenate([s_ref[...], y], axis=1)
    h1 = _inorm_relu_t(_conv9_t(z, wa0, wdim))
    h2 = _inorm_relu_t(_conv9_t(h1, wa1, wdim))
    res = jnp.stack([jnp.dot(ws[...], z[bi],
                             preferred_element_type=jnp.float32)
                     for bi in range(z.shape[0])], axis=0)
    o_ref[...] = h2 + res + bs[...][None]


def _final_body_t(u_ref, s_ref, lat_ref, wu, wa0, wa1, ws, bs, wo, bo,
                  o_ref, *, wdim):
    y = _inorm_relu_t(_conv9_t(u_ref[...], wu, wdim))
    z = jnp.concatenate([s_ref[...], y], axis=1)
    h1 = _inorm_relu_t(_conv9_t(z, wa0, wdim))
    h2 = _inorm_relu_t(_conv9_t(h1, wa1, wdim))
    res = jnp.stack([jnp.dot(ws[...], z[bi],
                             preferred_element_type=jnp.float32)
                     for bi in range(z.shape[0])], axis=0)
    u1 = h2 + res + bs[...][None]
    out = jnp.stack([jnp.dot(wo[...], u1[bi],
                             preferred_element_type=jnp.float32)
                     for bi in range(u1.shape[0])], axis=0)
    o_ref[...] = out + bo[...][None] + lat_ref[...]


def _spec_batch(shape, b):
    nd = len(shape)
    return pl.BlockSpec((b,) + tuple(shape[1:]),
                        lambda i, _n=nd: (i,) + (0,) * (_n - 1))


def _spec_full(shape):
    nd = len(shape)
    return pl.BlockSpec(tuple(shape), lambda i, _n=nd: (0,) * _n)


def _w9(w):
    return jnp.transpose(w, (2, 3, 1, 0)).reshape(9 * w.shape[1], w.shape[0])


def _w9t(w):
    return jnp.transpose(w, (0, 2, 3, 1)).reshape(w.shape[0], 9 * w.shape[1])


def _up2(x):
    for ax in (2, 3):
        n = x.shape[ax]
        lo = jnp.concatenate([lax.slice_in_dim(x, 0, 1, axis=ax),
                              lax.slice_in_dim(x, 0, n - 1, axis=ax)], axis=ax)
        hi = jnp.concatenate([lax.slice_in_dim(x, 1, n, axis=ax),
                              lax.slice_in_dim(x, n - 1, n, axis=ax)], axis=ax)
        ev = 0.75 * x + 0.25 * lo
        od = 0.75 * x + 0.25 * hi
        x = jnp.stack([ev, od], axis=ax + 1).reshape(
            x.shape[:ax] + (2 * n,) + x.shape[ax + 1:])
    return x


def _pick_b(n, pref):
    for b in range(min(pref, n), 0, -1):
        if n % b == 0:
            return b
    return 1


def _run(body, ins, batched, b, out_shapes, scratch=()):
    n = ins[0].shape[0]
    specs = [_spec_batch(a.shape, b) if k else _spec_full(a.shape)
             for a, k in zip(ins, batched)]
    out_specs = [_spec_batch(s.shape, b) for s in out_shapes]
    return pl.pallas_call(
        body,
        grid=(n // b,),
        in_specs=specs,
        out_shape=out_shapes,
        out_specs=out_specs,
        scratch_shapes=list(scratch),
        compiler_params=pltpu.CompilerParams(
            dimension_semantics=("parallel",)),
    )(*ins)


def kernel(image_latent, base_watermark, input_conv__w0, input_conv__b0, down1__w0, down1__w1, down1__w2, down1__b0, down1__b1, down1__b2, down2__w0, down2__w1, down2__w2, down2__b0, down2__b1, down2__b2, down3__w0, down3__w1, down3__w2, down3__b0, down3__b1, down3__b2, bottleneck__w0, bottleneck__w1, bottleneck__w2, bottleneck__b0, bottleneck__b1, bottleneck__b2, up3__w0, up3__b0, att3__w0, att3__w1, att3__b0, att3__b1, att3__ws, att3__bs, up2__w0, up2__b0, att2__w0, att2__w1, att2__b0, att2__b1, att2__ws, att2__bs, up1__w0, up1__b0, att1__w0, att1__w1, att1__b0, att1__b1, att1__ws, att1__bs, out__w, out__b):
    n, cl, h, w = image_latent.shape
    lat = image_latent.reshape(n, cl, h * w)
    wmk = base_watermark.reshape(n, cl, h * w)

    hd4 = input_conv__w0.shape[0]
    hd2 = down1__w1.shape[0]
    hd = down2__w1.shape[0]
    hd2x = down3__w1.shape[0]
    f32 = jnp.float32

    dws = [_w9(v) for v in
           (input_conv__w0, down1__w0, down1__w1, down1__w2,
            down2__w0, down2__w1, down2__w2,
            down3__w0, down3__w1, down3__w2,
            bottleneck__w0, bottleneck__w1, bottleneck__w2)]
    b1 = _pick_b(n, 2)
    scratch = [pltpu.VMEM((b1, h + 2, w + 2, hd4), f32),
               pltpu.VMEM((b1, h // 2 + 2, w // 2 + 2, hd2), f32),
               pltpu.VMEM((b1, h // 4 + 2, w // 4 + 2, hd), f32)]
    x0c, d1c, d2c, bnc = _run(
        _down_body, [lat, wmk] + dws, [True, True] + [False] * 13, b1,
        [jax.ShapeDtypeStruct((n, hd4, h * w), f32),
         jax.ShapeDtypeStruct((n, hd2, h * w // 4), f32),
         jax.ShapeDtypeStruct((n, hd, h * w // 16), f32),
         jax.ShapeDtypeStruct((n, hd2x, h * w // 64), f32)],
        scratch)

    def up_cm(xc, hh):
        c = xc.shape[1]
        u = _up2(xc.reshape(n, c, hh, hh))
        return u.reshape(n, c, 4 * hh * hh)

    def att_stage(up_in, skip, wu, wa0, wa1, ws, bs, cout, wdim, b):
        body = functools.partial(_att_body_t, wdim=wdim)
        ins = [up_in, skip, _w9t(wu), _w9t(wa0), _w9t(wa1),
               ws[:, :, 0, 0], bs.reshape(-1, 1)]
        return _run(body, ins, [True, True] + [False] * 5, b,
                    [jax.ShapeDtypeStruct((n, cout, wdim * wdim), f32)])[0]

    u3 = att_stage(up_cm(bnc, h // 8), d2c, up3__w0, att3__w0, att3__w1,
                   att3__ws, att3__bs, hd, h // 4, _pick_b(n, 8))
    u2 = att_stage(up_cm(u3, h // 4), d1c, up2__w0, att2__w0, att2__w1,
                   att2__ws, att2__bs, hd2, h // 2, _pick_b(n, 8))

    fbody = functools.partial(_final_body_t, wdim=w)
    fins = [up_cm(u2, h // 2), x0c, lat, _w9t(up1__w0), _w9t(att1__w0),
            _w9t(att1__w1), att1__ws[:, :, 0, 0], att1__bs.reshape(-1, 1),
            out__w[:, :, 0, 0], out__b.reshape(-1, 1)]
    out = _run(fbody, fins, [True, True, True] + [False] * 7,
               _pick_b(n, 2),
               [jax.ShapeDtypeStruct((n, out__w.shape[0], h * w), f32)])[0]
    return out.reshape(n, out__w.shape[0], h, w)


# speedup vs baseline: 7.5787x; 2.0465x over previous
import functools

import jax
import jax.numpy as jnp
from jax import lax
from jax.experimental import pallas as pl
from jax.experimental.pallas import tpu as pltpu

_EPS = 1e-5


def _inorm_relu_t(y):
    m = jnp.mean(y, axis=2, keepdims=True)
    v = jnp.mean(y * y, axis=2, keepdims=True) - m * m
    return jnp.maximum((y - m) * lax.rsqrt(v + _EPS), 0.0)


def _shift_p(x, off):
    if off == 0:
        return x
    p = x.shape[-1]
    if off > 0:
        z = jnp.zeros(x.shape[:-1] + (off,), x.dtype)
        return jnp.concatenate([x[..., off:], z], axis=-1)
    z = jnp.zeros(x.shape[:-1] + (-off,), x.dtype)
    return jnp.concatenate([z, x[..., :p + off]], axis=-1)


def _conv9_t(x, w9t, wdim):
    b, c, p = x.shape
    col = lax.broadcasted_iota(jnp.int32, (1, 1, p), 2) % wdim
    taps = []
    for di in range(3):
        for dj in range(3):
            t = _shift_p(x, (di - 1) * wdim + (dj - 1))
            if dj == 0:
                t = jnp.where(col > 0, t, 0.0)
            elif dj == 2:
                t = jnp.where(col < wdim - 1, t, 0.0)
            taps.append(t)
    lhs = jnp.concatenate(taps, axis=1)
    return jnp.stack([jnp.dot(w9t[...], lhs[bi],
                              preferred_element_type=jnp.float32)
                      for bi in range(b)], axis=0)


def _down_a_body(lat_ref, wm_ref, w_ic, wd10, x0c_ref, y1_ref, *, wdim):
    xc = jnp.concatenate([lat_ref[...], wm_ref[...]], axis=1)
    x0 = _inorm_relu_t(_conv9_t(xc, w_ic, wdim))
    x0c_ref[...] = x0
    y1_ref[...] = _conv9_t(x0, wd10, wdim)


def _down_mid_body(y_ref, w_l1, w_l2, w_next0, dc_ref, ynext_ref, *, wdim):
    y = _inorm_relu_t(y_ref[...])
    y = _inorm_relu_t(_conv9_t(y, w_l1, wdim))
    y = _inorm_relu_t(_conv9_t(y, w_l2, wdim))
    dc_ref[...] = y
    ynext_ref[...] = _conv9_t(y, w_next0, wdim)


def _down_bn_body(y_ref, w_l1, w_l2, wb0, wb1, wb2, bn_ref, *, wdim):
    y = _inorm_relu_t(y_ref[...])
    y = _inorm_relu_t(_conv9_t(y, w_l1, wdim))
    d3 = _inorm_relu_t(_conv9_t(y, w_l2, wdim))
    y = _inorm_relu_t(_conv9_t(d3, wb0, wdim))
    y = _inorm_relu_t(_conv9_t(y, wb1, wdim))
    y = _inorm_relu_t(_conv9_t(y, wb2, wdim))
    bn_ref[...] = y + d3


def _att_body_t(u_ref, s_ref, wu, wa0, wa1, ws, bs, o_ref, *, wdim):
    y = _inorm_relu_t(_conv9_t(u_ref[...], wu, wdim))
    z = jnp.concatenate([s_ref[...], y], axis=1)
    h1 = _inorm_relu_t(_conv9_t(z, wa0, wdim))
    h2 = _inorm_relu_t(_conv9_t(h1, wa1, wdim))
    res = jnp.stack([jnp.dot(ws[...], z[bi],
                             preferred_element_type=jnp.float32)
                     for bi in range(z.shape[0])], axis=0)
    o_ref[...] = h2 + res + bs[...][None]


def _final_body_t(u_ref, s_ref, lat_ref, wu, wa0, wa1, ws, bs, wo, bo,
                  o_ref, *, wdim):
    y = _inorm_relu_t(_conv9_t(u_ref[...], wu, wdim))
    z = jnp.concatenate([s_ref[...], y], axis=1)
    h1 = _inorm_relu_t(_conv9_t(z, wa0, wdim))
    h2 = _inorm_relu_t(_conv9_t(h1, wa1, wdim))
    res = jnp.stack([jnp.dot(ws[...], z[bi],
                             preferred_element_type=jnp.float32)
                     for bi in range(z.shape[0])], axis=0)
    u1 = h2 + res + bs[...][None]
    out = jnp.stack([jnp.dot(wo[...], u1[bi],
                             preferred_element_type=jnp.float32)
                     for bi in range(u1.shape[0])], axis=0)
    o_ref[...] = out + bo[...][None] + lat_ref[...]


def _spec_batch(shape, b):
    nd = len(shape)
    return pl.BlockSpec((b,) + tuple(shape[1:]),
                        lambda i, _n=nd: (i,) + (0,) * (_n - 1))


def _spec_full(shape):
    nd = len(shape)
    return pl.BlockSpec(tuple(shape), lambda i, _n=nd: (0,) * _n)


def _w9t(w):
    return jnp.transpose(w, (0, 2, 3, 1)).reshape(w.shape[0], 9 * w.shape[1])


def _up2(x):
    for ax in (2, 3):
        n = x.shape[ax]
        lo = jnp.concatenate([lax.slice_in_dim(x, 0, 1, axis=ax),
                              lax.slice_in_dim(x, 0, n - 1, axis=ax)], axis=ax)
        hi = jnp.concatenate([lax.slice_in_dim(x, 1, n, axis=ax),
                              lax.slice_in_dim(x, n - 1, n, axis=ax)], axis=ax)
        ev = 0.75 * x + 0.25 * lo
        od = 0.75 * x + 0.25 * hi
        x = jnp.stack([ev, od], axis=ax + 1).reshape(
            x.shape[:ax] + (2 * n,) + x.shape[ax + 1:])
    return x


def _pick_b(n, pref):
    for b in range(min(pref, n), 0, -1):
        if n % b == 0:
            return b
    return 1


def _run(body, ins, batched, b, out_shapes):
    n = ins[0].shape[0]
    specs = [_spec_batch(a.shape, b) if k else _spec_full(a.shape)
             for a, k in zip(ins, batched)]
    out_specs = [_spec_batch(s.shape, b) for s in out_shapes]
    return pl.pallas_call(
        body,
        grid=(n // b,),
        in_specs=specs,
        out_shape=out_shapes,
        out_specs=out_specs,
        compiler_params=pltpu.CompilerParams(
            dimension_semantics=("parallel",)),
    )(*ins)


def kernel(image_latent, base_watermark, input_conv__w0, input_conv__b0, down1__w0, down1__w1, down1__w2, down1__b0, down1__b1, down1__b2, down2__w0, down2__w1, down2__w2, down2__b0, down2__b1, down2__b2, down3__w0, down3__w1, down3__w2, down3__b0, down3__b1, down3__b2, bottleneck__w0, bottleneck__w1, bottleneck__w2, bottleneck__b0, bottleneck__b1, bottleneck__b2, up3__w0, up3__b0, att3__w0, att3__w1, att3__b0, att3__b1, att3__ws, att3__bs, up2__w0, up2__b0, att2__w0, att2__w1, att2__b0, att2__b1, att2__ws, att2__bs, up1__w0, up1__b0, att1__w0, att1__w1, att1__b0, att1__b1, att1__ws, att1__bs, out__w, out__b):
    n, cl, h, w = image_latent.shape
    lat = image_latent.reshape(n, cl, h * w)
    wmk = base_watermark.reshape(n, cl, h * w)

    hd4 = input_conv__w0.shape[0]
    hd2 = down1__w1.shape[0]
    hd = down2__w1.shape[0]
    hd2x = down3__w1.shape[0]
    f32 = jnp.float32

    def sub2(y, hh):
        c = y.shape[1]
        y4 = y.reshape(n, c, hh, hh)[:, :, ::2, ::2]
        return y4.reshape(n, c, hh * hh // 4)

    def sds(c, p):
        return jax.ShapeDtypeStruct((n, c, p), f32)

    body_a = functools.partial(_down_a_body, wdim=w)
    x0c, y1 = _run(body_a,
                   [lat, wmk, _w9t(input_conv__w0), _w9t(down1__w0)],
                   [True, True, False, False], _pick_b(n, 2),
                   [sds(hd4, h * w), sds(hd4, h * w)])

    body_b = functools.partial(_down_mid_body, wdim=w // 2)
    d1c, y2 = _run(body_b,
                   [sub2(y1, h), _w9t(down1__w1), _w9t(down1__w2),
                    _w9t(down2__w0)],
                   [True] + [False] * 3, _pick_b(n, 8),
                   [sds(hd2, h * w // 4), sds(hd2, h * w // 4)])

    body_c = functools.partial(_down_mid_body, wdim=w // 4)
    d2c, y3 = _run(body_c,
                   [sub2(y2, h // 2), _w9t(down2__w1), _w9t(down2__w2),
                    _w9t(down3__w0)],
                   [True] + [False] * 3, _pick_b(n, 16),
                   [sds(hd, h * w // 16), sds(hd, h * w // 16)])

    body_d = functools.partial(_down_bn_body, wdim=w // 8)
    bnc = _run(body_d,
               [sub2(y3, h // 4), _w9t(down3__w1), _w9t(down3__w2),
                _w9t(bottleneck__w0), _w9t(bottleneck__w1),
                _w9t(bottleneck__w2)],
               [True] + [False] * 5, _pick_b(n, 16),
               [sds(hd2x, h * w // 64)])[0]

    def up_cm(xc, hh):
        c = xc.shape[1]
        u = _up2(xc.reshape(n, c, hh, hh))
        return u.reshape(n, c, 4 * hh * hh)

    def att_stage(up_in, skip, wu, wa0, wa1, ws, bs, cout, wdim, b):
        body = functools.partial(_att_body_t, wdim=wdim)
        ins = [up_in, skip, _w9t(wu), _w9t(wa0), _w9t(wa1),
               ws[:, :, 0, 0], bs.reshape(-1, 1)]
        return _run(body, ins, [True, True] + [False] * 5, b,
                    [sds(cout, wdim * wdim)])[0]

    u3 = att_stage(up_cm(bnc, h // 8), d2c, up3__w0, att3__w0, att3__w1,
                   att3__ws, att3__bs, hd, h // 4, _pick_b(n, 8))
    u2 = att_stage(up_cm(u3, h // 4), d1c, up2__w0, att2__w0, att2__w1,
                   att2__ws, att2__bs, hd2, h // 2, _pick_b(n, 8))

    fbody = functools.partial(_final_body_t, wdim=w)
    fins = [up_cm(u2, h // 2), x0c, lat, _w9t(up1__w0), _w9t(att1__w0),
            _w9t(att1__w1), att1__ws[:, :, 0, 0], att1__bs.reshape(-1, 1),
            out__w[:, :, 0, 0], out__b.reshape(-1, 1)]
    out = _run(fbody, fins, [True, True, True] + [False] * 7,
               _pick_b(n, 2),
               [sds(out__w.shape[0], h * w)])[0]
    return out.reshape(n, out__w.shape[0], h, w)


# speedup vs baseline: 7.7389x; 1.0211x over previous
import functools

import jax
import jax.numpy as jnp
from jax import lax
from jax.experimental import pallas as pl
from jax.experimental.pallas import tpu as pltpu

_EPS = 1e-5


def _inorm_relu_t(y):
    m = jnp.mean(y, axis=2, keepdims=True)
    v = jnp.mean(y * y, axis=2, keepdims=True) - m * m
    return jnp.maximum((y - m) * lax.rsqrt(v + _EPS), 0.0)


def _shift_p(x, off):
    if off == 0:
        return x
    p = x.shape[-1]
    if off > 0:
        z = jnp.zeros(x.shape[:-1] + (off,), x.dtype)
        return jnp.concatenate([x[..., off:], z], axis=-1)
    z = jnp.zeros(x.shape[:-1] + (-off,), x.dtype)
    return jnp.concatenate([z, x[..., :p + off]], axis=-1)


def _conv9_t(x, w9t, wdim):
    b, c, p = x.shape
    col = lax.broadcasted_iota(jnp.int32, (1, 1, p), 2) % wdim
    taps = []
    for di in range(3):
        for dj in range(3):
            t = _shift_p(x, (di - 1) * wdim + (dj - 1))
            if dj == 0:
                t = jnp.where(col > 0, t, 0.0)
            elif dj == 2:
                t = jnp.where(col < wdim - 1, t, 0.0)
            taps.append(t)
    lhs = jnp.concatenate(taps, axis=1)
    return jnp.stack([jnp.dot(w9t[...], lhs[bi],
                              preferred_element_type=jnp.float32)
                      for bi in range(b)], axis=0)


def _down_a_body(lat_ref, wm_ref, w_ic, wd10, x0c_ref, y1_ref, *, wdim):
    xc = jnp.concatenate([lat_ref[...], wm_ref[...]], axis=1)
    x0 = _inorm_relu_t(_conv9_t(xc, w_ic, wdim))
    x0c_ref[...] = x0
    y1_ref[...] = _conv9_t(x0, wd10, wdim)


def _down_mid_body(y_ref, w_l1, w_l2, w_next0, dc_ref, ynext_ref, *, wdim):
    y = _inorm_relu_t(y_ref[...])
    y = _inorm_relu_t(_conv9_t(y, w_l1, wdim))
    y = _inorm_relu_t(_conv9_t(y, w_l2, wdim))
    dc_ref[...] = y
    ynext_ref[...] = _conv9_t(y, w_next0, wdim)


def _down_bn_body(y_ref, w_l1, w_l2, wb0, wb1, wb2, bn_ref, *, wdim):
    y = _inorm_relu_t(y_ref[...])
    y = _inorm_relu_t(_conv9_t(y, w_l1, wdim))
    d3 = _inorm_relu_t(_conv9_t(y, w_l2, wdim))
    y = _inorm_relu_t(_conv9_t(d3, wb0, wdim))
    y = _inorm_relu_t(_conv9_t(y, wb1, wdim))
    y = _inorm_relu_t(_conv9_t(y, wb2, wdim))
    bn_ref[...] = y + d3


def _att_body_t(u_ref, s_ref, wu, wa0, wa1, ws, bs, o_ref, *, wdim):
    y = _inorm_relu_t(_conv9_t(u_ref[...], wu, wdim))
    z = jnp.concatenate([s_ref[...], y], axis=1)
    h1 = _inorm_relu_t(_conv9_t(z, wa0, wdim))
    h2 = _inorm_relu_t(_conv9_t(h1, wa1, wdim))
    res = jnp.stack([jnp.dot(ws[...], z[bi],
                             preferred_element_type=jnp.float32)
                     for bi in range(z.shape[0])], axis=0)
    o_ref[...] = h2 + res + bs[...][None]


def _final_body_t(u_ref, s_ref, lat_ref, wu, wa0, wa1, ws, bs, wo, bo,
                  o_ref, *, wdim):
    y = _inorm_relu_t(_conv9_t(u_ref[...], wu, wdim))
    z = jnp.concatenate([s_ref[...], y], axis=1)
    h1 = _inorm_relu_t(_conv9_t(z, wa0, wdim))
    h2 = _inorm_relu_t(_conv9_t(h1, wa1, wdim))
    res = jnp.stack([jnp.dot(ws[...], z[bi],
                             preferred_element_type=jnp.float32)
                     for bi in range(z.shape[0])], axis=0)
    u1 = h2 + res + bs[...][None]
    out = jnp.stack([jnp.dot(wo[...], u1[bi],
                             preferred_element_type=jnp.float32)
                     for bi in range(u1.shape[0])], axis=0)
    o_ref[...] = out + bo[...][None] + lat_ref[...]


def _spec_batch(shape, b):
    nd = len(shape)
    return pl.BlockSpec((b,) + tuple(shape[1:]),
                        lambda i, _n=nd: (i,) + (0,) * (_n - 1))


def _spec_full(shape):
    nd = len(shape)
    return pl.BlockSpec(tuple(shape), lambda i, _n=nd: (0,) * _n)


def _w9t(w):
    return jnp.transpose(w, (0, 2, 3, 1)).reshape(w.shape[0], 9 * w.shape[1])


def _up2(x):
    for ax in (2, 3):
        n = x.shape[ax]
        lo = jnp.concatenate([lax.slice_in_dim(x, 0, 1, axis=ax),
                              lax.slice_in_dim(x, 0, n - 1, axis=ax)], axis=ax)
        hi = jnp.concatenate([lax.slice_in_dim(x, 1, n, axis=ax),
                              lax.slice_in_dim(x, n - 1, n, axis=ax)], axis=ax)
        ev = 0.75 * x + 0.25 * lo
        od = 0.75 * x + 0.25 * hi
        x = jnp.stack([ev, od], axis=ax + 1).reshape(
            x.shape[:ax] + (2 * n,) + x.shape[ax + 1:])
    return x


def _pick_b(n, pref):
    for b in range(min(pref, n), 0, -1):
        if n % b == 0:
            return b
    return 1


def _run(body, ins, batched, b, out_shapes):
    n = ins[0].shape[0]
    specs = [_spec_batch(a.shape, b) if k else _spec_full(a.shape)
             for a, k in zip(ins, batched)]
    out_specs = [_spec_batch(s.shape, b) for s in out_shapes]
    return pl.pallas_call(
        body,
        grid=(n // b,),
        in_specs=specs,
        out_shape=out_shapes,
        out_specs=out_specs,
        compiler_params=pltpu.CompilerParams(
            dimension_semantics=("parallel",)),
    )(*ins)


def kernel(image_latent, base_watermark, input_conv__w0, input_conv__b0, down1__w0, down1__w1, down1__w2, down1__b0, down1__b1, down1__b2, down2__w0, down2__w1, down2__w2, down2__b0, down2__b1, down2__b2, down3__w0, down3__w1, down3__w2, down3__b0, down3__b1, down3__b2, bottleneck__w0, bottleneck__w1, bottleneck__w2, bottleneck__b0, bottleneck__b1, bottleneck__b2, up3__w0, up3__b0, att3__w0, att3__w1, att3__b0, att3__b1, att3__ws, att3__bs, up2__w0, up2__b0, att2__w0, att2__w1, att2__b0, att2__b1, att2__ws, att2__bs, up1__w0, up1__b0, att1__w0, att1__w1, att1__b0, att1__b1, att1__ws, att1__bs, out__w, out__b):
    n, cl, h, w = image_latent.shape
    lat = image_latent.reshape(n, cl, h * w)
    wmk = base_watermark.reshape(n, cl, h * w)

    hd4 = input_conv__w0.shape[0]
    hd2 = down1__w1.shape[0]
    hd = down2__w1.shape[0]
    hd2x = down3__w1.shape[0]
    f32 = jnp.float32

    def sub2(y, hh):
        c = y.shape[1]
        y4 = y.reshape(n, c, hh, hh)[:, :, ::2, ::2]
        return y4.reshape(n, c, hh * hh // 4)

    def sds(c, p):
        return jax.ShapeDtypeStruct((n, c, p), f32)

    body_a = functools.partial(_down_a_body, wdim=w)
    x0c, y1 = _run(body_a,
                   [lat, wmk, _w9t(input_conv__w0), _w9t(down1__w0)],
                   [True, True, False, False], _pick_b(n, 4),
                   [sds(hd4, h * w), sds(hd4, h * w)])

    body_b = functools.partial(_down_mid_body, wdim=w // 2)
    d1c, y2 = _run(body_b,
                   [sub2(y1, h), _w9t(down1__w1), _w9t(down1__w2),
                    _w9t(down2__w0)],
                   [True] + [False] * 3, _pick_b(n, 8),
                   [sds(hd2, h * w // 4), sds(hd2, h * w // 4)])

    body_c = functools.partial(_down_mid_body, wdim=w // 4)
    d2c, y3 = _run(body_c,
                   [sub2(y2, h // 2), _w9t(down2__w1), _w9t(down2__w2),
                    _w9t(down3__w0)],
                   [True] + [False] * 3, _pick_b(n, 16),
                   [sds(hd, h * w // 16), sds(hd, h * w // 16)])

    body_d = functools.partial(_down_bn_body, wdim=w // 8)
    bnc = _run(body_d,
               [sub2(y3, h // 4), _w9t(down3__w1), _w9t(down3__w2),
                _w9t(bottleneck__w0), _w9t(bottleneck__w1),
                _w9t(bottleneck__w2)],
               [True] + [False] * 5, _pick_b(n, 16),
               [sds(hd2x, h * w // 64)])[0]

    def up_cm(xc, hh):
        c = xc.shape[1]
        u = _up2(xc.reshape(n, c, hh, hh))
        return u.reshape(n, c, 4 * hh * hh)

    def att_stage(up_in, skip, wu, wa0, wa1, ws, bs, cout, wdim, b):
        body = functools.partial(_att_body_t, wdim=wdim)
        ins = [up_in, skip, _w9t(wu), _w9t(wa0), _w9t(wa1),
               ws[:, :, 0, 0], bs.reshape(-1, 1)]
        return _run(body, ins, [True, True] + [False] * 5, b,
                    [sds(cout, wdim * wdim)])[0]

    u3 = att_stage(up_cm(bnc, h // 8), d2c, up3__w0, att3__w0, att3__w1,
                   att3__ws, att3__bs, hd, h // 4, _pick_b(n, 8))
    u2 = att_stage(up_cm(u3, h // 4), d1c, up2__w0, att2__w0, att2__w1,
                   att2__ws, att2__bs, hd2, h // 2, _pick_b(n, 8))

    fbody = functools.partial(_final_body_t, wdim=w)
    fins = [up_cm(u2, h // 2), x0c, lat, _w9t(up1__w0), _w9t(att1__w0),
            _w9t(att1__w1), att1__ws[:, :, 0, 0], att1__bs.reshape(-1, 1),
            out__w[:, :, 0, 0], out__b.reshape(-1, 1)]
    out = _run(fbody, fins, [True, True, True] + [False] * 7,
               _pick_b(n, 4),
               [sds(out__w.shape[0], h * w)])[0]
    return out.reshape(n, out__w.shape[0], h, w)
